# Initial kernel scaffold; baseline (speedup 1.0000x reference)
import functools

import jax
import jax.numpy as jnp
from jax import lax
from jax.experimental import pallas as pl
from jax.experimental.pallas import tpu as pltpu
from jax.experimental.pallas import tpu_sc as plsc

_PIECES = 16
_NUM_WORKERS = 32
_ROWS_PER_DMA = 64


def _make_shuffle(total_rows: int, chunk: int, samples_per_worker: int,
                  rows_per_sample: int):
    dmas_per_sample = rows_per_sample // _ROWS_PER_DMA
    mesh = plsc.VectorSubcoreMesh(core_axis_name="c", subcore_axis_name="s")

    @functools.partial(
        pl.kernel,
        mesh=mesh,
        out_type=jax.ShapeDtypeStruct((total_rows, chunk), jnp.float32),
        scratch_types=[
            pltpu.VMEM((_PIECES,), jnp.float32),
            pltpu.VMEM((dmas_per_sample, _ROWS_PER_DMA), jnp.int32),
            pltpu.VMEM((_ROWS_PER_DMA, chunk), jnp.float32),
            pltpu.SemaphoreType.DMA,
        ],
    )
    def shuffle(x_hbm, scores_hbm, out_hbm, scores_v, idx_v, buf_v, sem):
        wid = lax.axis_index("s") * 2 + lax.axis_index("c")
        groups_per_row = _ROWS_PER_DMA // _PIECES
        for i in range(samples_per_worker):
            n = wid * samples_per_worker + i
            pltpu.sync_copy(scores_hbm.at[n], scores_v)
            iota = lax.iota(jnp.int32, _PIECES)
            _, perm = plsc.sort_key_val(scores_v[...], iota)
            base = n * rows_per_sample
            for c in range(rows_per_sample // _PIECES):
                idx_v[c // groups_per_row,
                      pl.ds((c % groups_per_row) * _PIECES, _PIECES)] = (
                          base + c * _PIECES + perm)
            for j in range(dmas_per_sample):
                pltpu.async_copy(x_hbm.at[idx_v.at[j]], buf_v, sem).wait()
                pltpu.sync_copy(
                    buf_v, out_hbm.at[pl.ds(base + j * _ROWS_PER_DMA,
                                            _ROWS_PER_DMA)])

    return shuffle


def kernel(x):
    N, C, L = x.shape
    chunk = L // _PIECES
    total_rows = N * C * _PIECES
    rows_per_sample = C * _PIECES
    samples_per_worker = N // _NUM_WORKERS
    scores = jax.random.uniform(jax.random.key(42), (N, 1, _PIECES),
                                dtype=jnp.float32).reshape(N, _PIECES)
    x_rows = x.reshape(total_rows, chunk)
    out = _make_shuffle(total_rows, chunk, samples_per_worker,
                        rows_per_sample)(x_rows, scores)
    return out.reshape(N, C, L)


# speedup vs baseline: 5.8469x; 5.8469x over previous
import functools

import jax
import jax.numpy as jnp
from jax import lax
from jax.experimental import pallas as pl
from jax.experimental.pallas import tpu as pltpu
from jax.experimental.pallas import tpu_sc as plsc

_PIECES = 16
_NUM_WORKERS = 32
_ROWS_PER_DMA = 64


def _make_shuffle(total_rows: int, chunk: int, samples_per_worker: int,
                  rows_per_sample: int):
    dmas_per_sample = rows_per_sample // _ROWS_PER_DMA
    mesh = plsc.VectorSubcoreMesh(core_axis_name="c", subcore_axis_name="s")

    @functools.partial(
        pl.kernel,
        mesh=mesh,
        out_type=jax.ShapeDtypeStruct((total_rows, chunk), jnp.float32),
        scratch_types=[
            pltpu.VMEM((_PIECES,), jnp.float32),
            pltpu.VMEM((dmas_per_sample, _ROWS_PER_DMA), jnp.int32),
            pltpu.VMEM((_ROWS_PER_DMA, chunk), jnp.float32),
            pltpu.SemaphoreType.DMA,
        ],
        compiler_params=pltpu.CompilerParams(needs_layout_passes=False),
    )
    def shuffle(x_hbm, scores_hbm, out_hbm, scores_v, idx_v, buf_v, sem):
        wid = lax.axis_index("s") * 2 + lax.axis_index("c")
        groups_per_row = _ROWS_PER_DMA // _PIECES
        for i in range(samples_per_worker):
            n = wid * samples_per_worker + i
            pltpu.sync_copy(scores_hbm.at[n], scores_v)
            iota = lax.iota(jnp.int32, _PIECES)
            _, perm = plsc.sort_key_val(scores_v[...], iota)
            base = n * rows_per_sample
            for c in range(rows_per_sample // _PIECES):
                idx_v[c // groups_per_row,
                      pl.ds((c % groups_per_row) * _PIECES, _PIECES)] = (
                          base + c * _PIECES + perm)
            for j in range(dmas_per_sample):
                pltpu.async_copy(x_hbm.at[idx_v.at[j]], buf_v, sem).wait()
                pltpu.sync_copy(
                    buf_v, out_hbm.at[pl.ds(base + j * _ROWS_PER_DMA,
                                            _ROWS_PER_DMA)])

    return shuffle


def kernel(x):
    N, C, L = x.shape
    chunk = L // _PIECES
    total_rows = N * C * _PIECES
    rows_per_sample = C * _PIECES
    samples_per_worker = N // _NUM_WORKERS
    scores = jax.random.uniform(jax.random.key(42), (N, 1, _PIECES),
                                dtype=jnp.float32).reshape(N, _PIECES)
    x_rows = x.reshape(total_rows, chunk)
    out = _make_shuffle(total_rows, chunk, samples_per_worker,
                        rows_per_sample)(x_rows, scores)
    return out.reshape(N, C, L)


# speedup vs baseline: 6.2049x; 1.0612x over previous
import functools

import jax
import jax.numpy as jnp
from jax import lax
from jax.experimental import pallas as pl
from jax.experimental.pallas import tpu as pltpu
from jax.experimental.pallas import tpu_sc as plsc

_PIECES = 16
_NUM_WORKERS = 32
_ROWS_PER_DMA = 64


def _make_shuffle(total_rows: int, chunk: int, samples_per_worker: int,
                  rows_per_sample: int):
    dmas_per_sample = rows_per_sample // _ROWS_PER_DMA
    total_dmas = samples_per_worker * dmas_per_sample
    mesh = plsc.VectorSubcoreMesh(core_axis_name="c", subcore_axis_name="s")

    @functools.partial(
        pl.kernel,
        mesh=mesh,
        out_type=jax.ShapeDtypeStruct((total_rows, chunk), jnp.float32),
        scratch_types=[
            pltpu.VMEM((_PIECES,), jnp.float32),
            pltpu.VMEM((total_dmas, _ROWS_PER_DMA), jnp.int32),
            pltpu.VMEM((2, _ROWS_PER_DMA, chunk), jnp.float32),
            pltpu.SemaphoreType.DMA,
            pltpu.SemaphoreType.DMA,
            pltpu.SemaphoreType.DMA,
            pltpu.SemaphoreType.DMA,
        ],
        compiler_params=pltpu.CompilerParams(needs_layout_passes=False),
    )
    def shuffle(x_hbm, scores_hbm, out_hbm, scores_v, idx_v, buf_v,
                gsem0, gsem1, ssem0, ssem1):
        wid = lax.axis_index("s") * 2 + lax.axis_index("c")
        gsem = (gsem0, gsem1)
        ssem = (ssem0, ssem1)
        for i in range(samples_per_worker):
            n = wid * samples_per_worker + i
            pltpu.sync_copy(scores_hbm.at[n], scores_v)
            iota = lax.iota(jnp.int32, _PIECES)
            _, perm = plsc.sort_key_val(scores_v[...], iota)
            base = n * rows_per_sample
            for c in range(rows_per_sample // _PIECES):
                flat = i * rows_per_sample + c * _PIECES
                idx_v[flat // _ROWS_PER_DMA,
                      pl.ds(flat % _ROWS_PER_DMA, _PIECES)] = (
                          base + c * _PIECES + perm)
        out_base = wid * samples_per_worker * rows_per_sample
        gathers = [None, None]
        scatters = [None, None]
        for t in range(total_dmas):
            b = t & 1
            if t >= 2:
                scatters[b].wait()
            gathers[b] = pltpu.async_copy(
                x_hbm.at[idx_v.at[t]], buf_v.at[b], gsem[b])
            if t >= 1:
                gathers[1 - b].wait()
                scatters[1 - b] = pltpu.async_copy(
                    buf_v.at[1 - b],
                    out_hbm.at[pl.ds(out_base + (t - 1) * _ROWS_PER_DMA,
                                     _ROWS_PER_DMA)],
                    ssem[1 - b])
        last = total_dmas - 1
        b = last & 1
        gathers[b].wait()
        scatters[b] = pltpu.async_copy(
            buf_v.at[b],
            out_hbm.at[pl.ds(out_base + last * _ROWS_PER_DMA,
                             _ROWS_PER_DMA)],
            ssem[b])
        scatters[1 - b].wait()
        scatters[b].wait()

    return shuffle


def kernel(x):
    N, C, L = x.shape
    chunk = L // _PIECES
    total_rows = N * C * _PIECES
    rows_per_sample = C * _PIECES
    samples_per_worker = N // _NUM_WORKERS
    scores = jax.random.uniform(jax.random.key(42), (N, 1, _PIECES),
                                dtype=jnp.float32).reshape(N, _PIECES)
    x_rows = x.reshape(total_rows, chunk)
    out = _make_shuffle(total_rows, chunk, samples_per_worker,
                        rows_per_sample)(x_rows, scores)
    return out.reshape(N, C, L)


# speedup vs baseline: 22.1702x; 3.5730x over previous
import functools

import jax
import jax.numpy as jnp
from jax import lax
from jax.experimental import pallas as pl
from jax.experimental.pallas import tpu as pltpu
from jax.experimental.pallas import tpu_sc as plsc

_PIECES = 16
_NUM_WORKERS = 32
_NBUF = 3


def _make_shuffle(N: int, C: int, L: int, chunk: int,
                  samples_per_worker: int):
    mesh = plsc.VectorSubcoreMesh(core_axis_name="c", subcore_axis_name="s")

    @functools.partial(
        pl.kernel,
        mesh=mesh,
        out_type=jax.ShapeDtypeStruct((N, C, L), jnp.float32),
        scratch_types=[
            pltpu.VMEM((_PIECES,), jnp.float32),
            pltpu.VMEM((_NBUF, C, chunk), jnp.float32),
            [pltpu.SemaphoreType.DMA] * _NBUF,
            [pltpu.SemaphoreType.DMA] * _NBUF,
        ],
        compiler_params=pltpu.CompilerParams(needs_layout_passes=False),
    )
    def shuffle(x_hbm, scores_hbm, out_hbm, scores_v,
                bufs, gsems, ssems):
        wid = lax.axis_index("s") * 2 + lax.axis_index("c")
        offsets = []
        for i in range(samples_per_worker):
            n = wid * samples_per_worker + i
            pltpu.sync_copy(scores_hbm.at[n], scores_v)
            iota = lax.iota(jnp.int32, _PIECES)
            _, perm = plsc.sort_key_val(scores_v[...], iota)
            offsets.append(perm * chunk)
        total = samples_per_worker * _PIECES
        gathers = [None] * _NBUF
        scatters = [None] * _NBUF
        for t in range(total):
            i, k = t // _PIECES, t % _PIECES
            n = wid * samples_per_worker + i
            r = t % _NBUF
            if t >= _NBUF:
                scatters[r].wait()
            gathers[r] = pltpu.async_copy(
                x_hbm.at[n, :, pl.ds(pl.multiple_of(offsets[i][k], chunk),
                                     chunk)],
                bufs.at[r], gsems[r])
            if t >= 1:
                tp = t - 1
                rp = tp % _NBUF
                ip, kp = tp // _PIECES, tp % _PIECES
                gathers[rp].wait()
                scatters[rp] = pltpu.async_copy(
                    bufs.at[rp],
                    out_hbm.at[wid * samples_per_worker + ip, :,
                               pl.ds(kp * chunk, chunk)],
                    ssems[rp])
        last = total - 1
        rl = last % _NBUF
        gathers[rl].wait()
        scatters[rl] = pltpu.async_copy(
            bufs.at[rl],
            out_hbm.at[wid * samples_per_worker + last // _PIECES, :,
                       pl.ds((last % _PIECES) * chunk, chunk)],
            ssems[rl])
        for r in range(_NBUF):
            scatters[r].wait()

    return shuffle


def kernel(x):
    N, C, L = x.shape
    chunk = L // _PIECES
    samples_per_worker = N // _NUM_WORKERS
    scores = jax.random.uniform(jax.random.key(42), (N, 1, _PIECES),
                                dtype=jnp.float32).reshape(N, _PIECES)
    return _make_shuffle(N, C, L, chunk, samples_per_worker)(x, scores)


# speedup vs baseline: 22.4695x; 1.0135x over previous
import functools

import jax
import jax.numpy as jnp
from jax import lax
from jax.experimental import pallas as pl
from jax.experimental.pallas import tpu as pltpu
from jax.experimental.pallas import tpu_sc as plsc

_PIECES = 16
_NUM_WORKERS = 32
_NBUF = 4


def _make_shuffle(N: int, C: int, L: int, chunk: int,
                  samples_per_worker: int):
    mesh = plsc.VectorSubcoreMesh(core_axis_name="c", subcore_axis_name="s")

    @functools.partial(
        pl.kernel,
        mesh=mesh,
        out_type=jax.ShapeDtypeStruct((N, C, L), jnp.float32),
        scratch_types=[
            pltpu.VMEM((samples_per_worker, _PIECES), jnp.float32),
            pltpu.VMEM((_NBUF, C, chunk), jnp.float32),
            [pltpu.SemaphoreType.DMA] * _NBUF,
            [pltpu.SemaphoreType.DMA] * _NBUF,
        ],
        compiler_params=pltpu.CompilerParams(needs_layout_passes=False),
    )
    def shuffle(x_hbm, scores_hbm, out_hbm, scores_v,
                bufs, gsems, ssems):
        wid = lax.axis_index("s") * 2 + lax.axis_index("c")
        n0 = wid * samples_per_worker
        pltpu.sync_copy(scores_hbm.at[pl.ds(n0, samples_per_worker)],
                        scores_v)
        offsets = []
        for i in range(samples_per_worker):
            iota = lax.iota(jnp.int32, _PIECES)
            _, perm = plsc.sort_key_val(scores_v[i], iota)
            offsets.append(perm * chunk)
        total = samples_per_worker * _PIECES
        gathers = [None] * _NBUF
        scatters = [None] * _NBUF
        for t in range(total):
            i, k = t // _PIECES, t % _PIECES
            n = wid * samples_per_worker + i
            r = t % _NBUF
            if t >= _NBUF:
                scatters[r].wait()
            gathers[r] = pltpu.async_copy(
                x_hbm.at[n, :, pl.ds(pl.multiple_of(offsets[i][k], chunk),
                                     chunk)],
                bufs.at[r], gsems[r])
            if t >= 1:
                tp = t - 1
                rp = tp % _NBUF
                ip, kp = tp // _PIECES, tp % _PIECES
                gathers[rp].wait()
                scatters[rp] = pltpu.async_copy(
                    bufs.at[rp],
                    out_hbm.at[wid * samples_per_worker + ip, :,
                               pl.ds(kp * chunk, chunk)],
                    ssems[rp])
        last = total - 1
        rl = last % _NBUF
        gathers[rl].wait()
        scatters[rl] = pltpu.async_copy(
            bufs.at[rl],
            out_hbm.at[wid * samples_per_worker + last // _PIECES, :,
                       pl.ds((last % _PIECES) * chunk, chunk)],
            ssems[rl])
        for r in range(_NBUF):
            scatters[r].wait()

    return shuffle


def kernel(x):
    N, C, L = x.shape
    chunk = L // _PIECES
    samples_per_worker = N // _NUM_WORKERS
    scores = jax.random.uniform(jax.random.key(42), (N, 1, _PIECES),
                                dtype=jnp.float32).reshape(N, _PIECES)
    return _make_shuffle(N, C, L, chunk, samples_per_worker)(x, scores)


# speedup vs baseline: 22.5374x; 1.0030x over previous
import functools

import jax
import jax.numpy as jnp
from jax import lax
from jax.experimental import pallas as pl
from jax.experimental.pallas import tpu as pltpu
from jax.experimental.pallas import tpu_sc as plsc

_PIECES = 16
_NUM_WORKERS = 32
_NBUF = 6


def _make_shuffle(N: int, C: int, L: int, chunk: int,
                  samples_per_worker: int):
    mesh = plsc.VectorSubcoreMesh(core_axis_name="c", subcore_axis_name="s")

    @functools.partial(
        pl.kernel,
        mesh=mesh,
        out_type=jax.ShapeDtypeStruct((N, C, L), jnp.float32),
        scratch_types=[
            pltpu.VMEM((samples_per_worker, _PIECES), jnp.float32),
            pltpu.VMEM((_NBUF, C, chunk), jnp.float32),
            [pltpu.SemaphoreType.DMA] * _NBUF,
            [pltpu.SemaphoreType.DMA] * _NBUF,
        ],
        compiler_params=pltpu.CompilerParams(needs_layout_passes=False),
    )
    def shuffle(x_hbm, scores_hbm, out_hbm, scores_v,
                bufs, gsems, ssems):
        wid = lax.axis_index("s") * 2 + lax.axis_index("c")
        n0 = wid * samples_per_worker
        pltpu.sync_copy(scores_hbm.at[pl.ds(n0, samples_per_worker)],
                        scores_v)
        offsets = []
        for i in range(samples_per_worker):
            iota = lax.iota(jnp.int32, _PIECES)
            _, perm = plsc.sort_key_val(scores_v[i], iota)
            offsets.append(perm * chunk)
        total = samples_per_worker * _PIECES
        gathers = [None] * _NBUF
        scatters = [None] * _NBUF
        for t in range(total):
            i, k = t // _PIECES, t % _PIECES
            n = wid * samples_per_worker + i
            r = t % _NBUF
            if t >= _NBUF:
                scatters[r].wait()
            gathers[r] = pltpu.async_copy(
                x_hbm.at[n, :, pl.ds(pl.multiple_of(offsets[i][k], chunk),
                                     chunk)],
                bufs.at[r], gsems[r])
            if t >= 1:
                tp = t - 1
                rp = tp % _NBUF
                ip, kp = tp // _PIECES, tp % _PIECES
                gathers[rp].wait()
                scatters[rp] = pltpu.async_copy(
                    bufs.at[rp],
                    out_hbm.at[wid * samples_per_worker + ip, :,
                               pl.ds(kp * chunk, chunk)],
                    ssems[rp])
        last = total - 1
        rl = last % _NBUF
        gathers[rl].wait()
        scatters[rl] = pltpu.async_copy(
            bufs.at[rl],
            out_hbm.at[wid * samples_per_worker + last // _PIECES, :,
                       pl.ds((last % _PIECES) * chunk, chunk)],
            ssems[rl])
        for r in range(_NBUF):
            scatters[r].wait()

    return shuffle


def kernel(x):
    N, C, L = x.shape
    chunk = L // _PIECES
    samples_per_worker = N // _NUM_WORKERS
    scores = jax.random.uniform(jax.random.key(42), (N, 1, _PIECES),
                                dtype=jnp.float32).reshape(N, _PIECES)
    return _make_shuffle(N, C, L, chunk, samples_per_worker)(x, scores)
